# baseline (speedup 1.0000x reference)
import functools

import jax
import jax.numpy as jnp
from jax import lax
from jax.experimental import pallas as pl
from jax.experimental.pallas import tpu as pltpu


def _pick_block(n, pref):
    for b in (pref, 32, 16, 8, 4, 2, 1):
        if n % b == 0:
            return b
    return 1


def _conv_head_kernel(col1_ref, w1_ref, b1_ref, w2_ref, b2_ref, w3_ref, b3_ref,
                      wd_ref, bd_ref, wl_ref, bl_ref,
                      x3_ref, th_ref, p2_ref, col2_ref, p3_ref, col3_ref,
                      *, Nb):
    f32 = jnp.float32

    x1 = col1_ref[...].reshape(Nb * 256, 32)
    o1 = jnp.maximum(
        jnp.dot(x1, w1_ref[...], preferred_element_type=f32) + b1_ref[...], 0.0)
    o1r = o1.reshape(Nb, 4, 4, 4, 4, 32)

    for a in range(4):
        zr = 0 if a == 0 else 4
        p2_ref[:, a * 5 + zr:a * 5 + zr + 1, :, :] = jnp.zeros((Nb, 1, 5, 128), f32)
    for b in range(4):
        zc = 0 if b == 0 else 4
        p2_ref[:, :, zc:zc + 1, b * 32:(b + 1) * 32] = jnp.zeros((Nb, 20, 1, 32), f32)
    for a in range(4):
        rh = (a + 3) % 4
        ro = 1 if a == 0 else 0
        for b in range(4):
            rw = (b + 3) % 4
            co = 1 if b == 0 else 0
            p2_ref[:, a * 5 + ro:a * 5 + ro + 4, co:co + 4,
                   b * 32:(b + 1) * 32] = o1r[:, rh, rw]

    x2 = p2_ref[...]
    for kh in range(3):
        for kw in range(3):
            t = kh * 3 + kw
            for ip in range(2):
                for jp in range(2):
                    q = ip * 2 + jp
                    ua, um = (2 * ip + kh) % 4, (2 * ip + kh) // 4
                    vb, vm = (2 * jp + kw) % 4, (2 * jp + kw) // 4
                    tap = x2[:, ua * 5 + um:ua * 5 + um + 4, vm:vm + 4,
                             vb * 32:(vb + 1) * 32]
                    col2_ref[:, q * 16:(q + 1) * 16,
                             32 * t:32 * (t + 1)] = tap.reshape(Nb, 16, 32)
    o2 = jnp.maximum(
        jnp.dot(col2_ref[...].reshape(Nb * 64, 288), w2_ref[...],
                preferred_element_type=f32) + b2_ref[...], 0.0)
    o2r = o2.reshape(Nb, 2, 2, 4, 4, 64)

    p3_ref[:, 0:1, :, :] = jnp.zeros((Nb, 1, 5, 128), f32)
    p3_ref[:, 9:10, :, :] = jnp.zeros((Nb, 1, 5, 128), f32)
    p3_ref[:, :, 0:1, 0:64] = jnp.zeros((Nb, 10, 1, 64), f32)
    p3_ref[:, :, 4:5, 64:128] = jnp.zeros((Nb, 10, 1, 64), f32)
    for ip in range(2):
        a = (ip + 1) % 2
        ro = (ip + 1) // 2
        for jp in range(2):
            b = (jp + 1) % 2
            co = (jp + 1) // 2
            p3_ref[:, a * 5 + ro:a * 5 + ro + 4, co:co + 4,
                   b * 64:(b + 1) * 64] = o2r[:, ip, jp]

    x3 = p3_ref[...]
    for kh in range(3):
        for kw in range(3):
            t = kh * 3 + kw
            tap = x3[:, (kh % 2) * 5 + kh // 2:(kh % 2) * 5 + kh // 2 + 4,
                     kw // 2:kw // 2 + 4, (kw % 2) * 64:(kw % 2 + 1) * 64]
            col3_ref[:, :, 64 * t:64 * (t + 1)] = tap.reshape(Nb, 16, 64)
    o3 = jnp.maximum(
        jnp.dot(col3_ref[...].reshape(Nb * 16, 576), w3_ref[...],
                preferred_element_type=f32) + b3_ref[...], 0.0)
    o3 = o3.reshape(Nb, 16, 128)
    x3_ref[...] = o3

    acc = bd_ref[...]
    for p in range(16):
        acc = acc + jnp.dot(o3[:, p, :], wd_ref[p], preferred_element_type=f32)
    h = jnp.maximum(acc, 0.0)
    th_ref[...] = jnp.dot(h, wl_ref[...], preferred_element_type=f32) + bl_ref[...]


def _warp_kernel(img_ref, th_ref, o_ref, *, Nb):
    f32 = jnp.float32
    pidx = lax.broadcasted_iota(jnp.int32, (1, 1024), 1).astype(f32)
    ohf = jnp.floor(pidx * (1.0 / 32.0))
    owf = pidx - 32.0 * ohf
    xn = (2.0 * owf + 1.0) * (1.0 / 32.0) - 1.0
    yn = (2.0 * ohf + 1.0) * (1.0 / 32.0) - 1.0
    xio = lax.broadcasted_iota(jnp.int32, (32, 1024), 0).astype(f32)
    rr = lax.broadcasted_iota(jnp.int32, (8, 96), 0)
    kk = lax.broadcasted_iota(jnp.int32, (8, 96), 1)
    R = (rr == kk // 32).astype(f32)

    for i in range(Nb):
        trow = th_ref[i]
        gx = trow[:, 0:1] * xn + trow[:, 1:2] * yn + trow[:, 2:3]
        gy = trow[:, 3:4] * xn + trow[:, 4:5] * yn + trow[:, 5:6]
        ix = gx * 16.0 + 15.5
        iy = gy * 16.0 + 15.5
        Bh = jnp.maximum(1.0 - jnp.abs(ix - xio), 0.0)
        Ah = jnp.maximum(1.0 - jnp.abs(iy - xio), 0.0)
        G = jnp.dot(img_ref[i], Bh, preferred_element_type=f32)
        Z = G.reshape(3, 32, 1024) * Ah[None, :, :]
        W = jnp.dot(R, Z.reshape(96, 1024), preferred_element_type=f32)
        o_ref[i] = W[0:3, :]


def kernel(image, w1, b1, w2, b2, w3, b3, wd, bd, wl, bl):
    f32 = jnp.float32
    N = image.shape[0]
    Nb1 = _pick_block(N, 64)
    Nb2 = _pick_block(N, 32)

    image = image.astype(f32)

    xp = jnp.pad(image, ((0, 0), (0, 0), (1, 1), (1, 1)))
    taps = [xp[:, :, kh:kh + 32:2, kw:kw + 32:2]
            for kh in range(3) for kw in range(3)]
    col1 = jnp.concatenate(taps + [jnp.zeros((N, 5, 16, 16), f32)], axis=1)
    col1 = (col1.reshape(N, 32, 4, 4, 4, 4)
            .transpose(0, 3, 5, 2, 4, 1).reshape(N, 256, 32))

    w1f = jnp.pad(w1.transpose(2, 3, 1, 0).reshape(27, 32), ((0, 5), (0, 0)))
    w2f = w2.transpose(2, 3, 1, 0).reshape(288, 64).astype(f32)
    w3f = w3.transpose(2, 3, 1, 0).reshape(576, 128).astype(f32)
    wdr = wd.reshape(128, 128, 16).transpose(2, 1, 0).astype(f32)
    w2h = wl.T.astype(f32)
    b1r = b1.reshape(1, 32).astype(f32)
    b2r = b2.reshape(1, 64).astype(f32)
    b3r = b3.reshape(1, 128).astype(f32)
    bdr = bd.reshape(1, 128).astype(f32)
    blr = bl.reshape(1, 6).astype(f32)

    conv_fn = functools.partial(_conv_head_kernel, Nb=Nb1)
    x3hwc, theta = pl.pallas_call(
        conv_fn,
        out_shape=(jax.ShapeDtypeStruct((N, 16, 128), f32),
                   jax.ShapeDtypeStruct((N, 6), f32)),
        grid_spec=pltpu.PrefetchScalarGridSpec(
            num_scalar_prefetch=0,
            grid=(N // Nb1,),
            in_specs=[
                pl.BlockSpec((Nb1, 256, 32), lambda n: (n, 0, 0)),
                pl.BlockSpec((32, 32), lambda n: (0, 0)),
                pl.BlockSpec((1, 32), lambda n: (0, 0)),
                pl.BlockSpec((288, 64), lambda n: (0, 0)),
                pl.BlockSpec((1, 64), lambda n: (0, 0)),
                pl.BlockSpec((576, 128), lambda n: (0, 0)),
                pl.BlockSpec((1, 128), lambda n: (0, 0)),
                pl.BlockSpec((16, 128, 128), lambda n: (0, 0, 0)),
                pl.BlockSpec((1, 128), lambda n: (0, 0)),
                pl.BlockSpec((128, 6), lambda n: (0, 0)),
                pl.BlockSpec((1, 6), lambda n: (0, 0)),
            ],
            out_specs=(pl.BlockSpec((Nb1, 16, 128), lambda n: (n, 0, 0)),
                       pl.BlockSpec((Nb1, 6), lambda n: (n, 0))),
            scratch_shapes=[
                pltpu.VMEM((Nb1, 20, 5, 128), f32),
                pltpu.VMEM((Nb1, 64, 288), f32),
                pltpu.VMEM((Nb1, 10, 5, 128), f32),
                pltpu.VMEM((Nb1, 16, 576), f32),
            ],
        ),
        compiler_params=pltpu.CompilerParams(
            dimension_semantics=("parallel",)),
    )(col1, w1f, b1r, w2f, b2r, w3f, b3r, wdr, bdr, w2h, blr)

    x_out = x3hwc.transpose(0, 2, 1).reshape(N, 128, 4, 4)

    warp_fn = functools.partial(_warp_kernel, Nb=Nb2)
    img96 = image.reshape(N, 96, 32)
    th3 = theta.reshape(N, 1, 6)
    warped = pl.pallas_call(
        warp_fn,
        out_shape=jax.ShapeDtypeStruct((N, 3, 1024), f32),
        grid_spec=pltpu.PrefetchScalarGridSpec(
            num_scalar_prefetch=0,
            grid=(N // Nb2,),
            in_specs=[
                pl.BlockSpec((Nb2, 96, 32), lambda n: (n, 0, 0)),
                pl.BlockSpec((Nb2, 1, 6), lambda n: (n, 0, 0)),
            ],
            out_specs=pl.BlockSpec((Nb2, 3, 1024), lambda n: (n, 0, 0)),
        ),
        compiler_params=pltpu.CompilerParams(
            dimension_semantics=("parallel",)),
    )(img96, th3)

    return warped.reshape(N, 3, 32, 32), x_out, theta


# speedup vs baseline: 1.1597x; 1.1597x over previous
import functools

import jax
import jax.numpy as jnp
from jax import lax
from jax.experimental import pallas as pl
from jax.experimental.pallas import tpu as pltpu


def _pick_block(n, pref):
    for b in (pref, 32, 16, 8, 4, 2, 1):
        if n % b == 0:
            return b
    return 1


def _conv_head_kernel(col1_ref, w1_ref, b1_ref, w2_ref, b2_ref, w3_ref, b3_ref,
                      wd_ref, bd_ref, wl_ref, bl_ref,
                      x3_ref, th_ref, p2_ref, col2_ref, p3_ref, col3_ref,
                      *, Nb):
    f32 = jnp.float32

    x1 = col1_ref[...].reshape(Nb * 256, 32)
    o1 = jnp.maximum(
        jnp.dot(x1, w1_ref[...], preferred_element_type=f32) + b1_ref[...], 0.0)
    o1r = o1.reshape(Nb, 4, 4, 4, 4, 32)

    zc32 = jnp.zeros((Nb, 4, 1, 32), f32)
    zr128 = jnp.zeros((Nb, 1, 5, 128), f32)
    for a in range(4):
        rh = (a + 3) % 4
        groups = []
        for b in range(4):
            rw = (b + 3) % 4
            blk = o1r[:, rh, rw]
            if b == 0:
                groups.append(jnp.concatenate([zc32, blk], axis=2))
            else:
                groups.append(jnp.concatenate([blk, zc32], axis=2))
        row4 = jnp.concatenate(groups, axis=3)
        if a == 0:
            plane = jnp.concatenate([zr128, row4], axis=1)
        else:
            plane = jnp.concatenate([row4, zr128], axis=1)
        p2_ref[:, a * 5:(a + 1) * 5, :, :] = plane

    x2 = p2_ref[...]

    def _tap2(t, ip, jp):
        kh, kw = t // 3, t % 3
        ua, um = (2 * ip + kh) % 4, (2 * ip + kh) // 4
        vb, vm = (2 * jp + kw) % 4, (2 * jp + kw) // 4
        return x2[:, ua * 5 + um:ua * 5 + um + 4, vm:vm + 4,
                  vb * 32:(vb + 1) * 32].reshape(Nb, 16, 32)

    for ip in range(2):
        for jp in range(2):
            q = ip * 2 + jp
            for tg in range(3):
                ts = range(4 * tg, min(4 * tg + 4, 9))
                blk = jnp.concatenate([_tap2(t, ip, jp) for t in ts], axis=2)
                col2_ref[:, q * 16:(q + 1) * 16,
                         128 * tg:128 * tg + 32 * len(ts)] = blk
    o2 = jnp.maximum(
        jnp.dot(col2_ref[...].reshape(Nb * 64, 288), w2_ref[...],
                preferred_element_type=f32) + b2_ref[...], 0.0)
    o2r = o2.reshape(Nb, 2, 2, 4, 4, 64)

    zc64 = jnp.zeros((Nb, 4, 1, 64), f32)
    for a in range(2):
        ip = 1 - a
        b0 = jnp.concatenate([zc64, o2r[:, ip, 1]], axis=2)
        b1 = jnp.concatenate([o2r[:, ip, 0], zc64], axis=2)
        row4 = jnp.concatenate([b0, b1], axis=3)
        if a == 0:
            plane = jnp.concatenate([zr128, row4], axis=1)
        else:
            plane = jnp.concatenate([row4, zr128], axis=1)
        p3_ref[:, a * 5:(a + 1) * 5, :, :] = plane

    x3 = p3_ref[...]

    def _tap3(t):
        kh, kw = t // 3, t % 3
        return x3[:, (kh % 2) * 5 + kh // 2:(kh % 2) * 5 + kh // 2 + 4,
                  kw // 2:kw // 2 + 4,
                  (kw % 2) * 64:(kw % 2 + 1) * 64].reshape(Nb, 16, 64)

    for tg in range(5):
        ts = range(2 * tg, min(2 * tg + 2, 9))
        blk = jnp.concatenate([_tap3(t) for t in ts], axis=2)
        col3_ref[:, :, 128 * tg:128 * tg + 64 * len(ts)] = blk
    o3 = jnp.maximum(
        jnp.dot(col3_ref[...].reshape(Nb * 16, 576), w3_ref[...],
                preferred_element_type=f32) + b3_ref[...], 0.0)
    o3 = o3.reshape(Nb, 16, 128)
    x3_ref[...] = o3

    acc = bd_ref[...]
    for p in range(16):
        acc = acc + jnp.dot(o3[:, p, :], wd_ref[p], preferred_element_type=f32)
    h = jnp.maximum(acc, 0.0)
    th_ref[...] = jnp.dot(h, wl_ref[...], preferred_element_type=f32) + bl_ref[...]


def _warp_kernel(img_ref, th_ref, o_ref, *, Nb):
    f32 = jnp.float32
    pidx = lax.broadcasted_iota(jnp.int32, (1, 1024), 1).astype(f32)
    ohf = jnp.floor(pidx * (1.0 / 32.0))
    owf = pidx - 32.0 * ohf
    xn = (2.0 * owf + 1.0) * (1.0 / 32.0) - 1.0
    yn = (2.0 * ohf + 1.0) * (1.0 / 32.0) - 1.0
    xio = lax.broadcasted_iota(jnp.int32, (32, 1024), 0).astype(f32)
    rr = lax.broadcasted_iota(jnp.int32, (8, 96), 0)
    kk = lax.broadcasted_iota(jnp.int32, (8, 96), 1)
    R = (rr == kk // 32).astype(f32)

    for i in range(Nb):
        trow = th_ref[i]
        gx = trow[:, 0:1] * xn + trow[:, 1:2] * yn + trow[:, 2:3]
        gy = trow[:, 3:4] * xn + trow[:, 4:5] * yn + trow[:, 5:6]
        ix = gx * 16.0 + 15.5
        iy = gy * 16.0 + 15.5
        Bh = jnp.maximum(1.0 - jnp.abs(ix - xio), 0.0)
        Ah = jnp.maximum(1.0 - jnp.abs(iy - xio), 0.0)
        G = jnp.dot(img_ref[i], Bh, preferred_element_type=f32)
        Z = G.reshape(3, 32, 1024) * Ah[None, :, :]
        W = jnp.dot(R, Z.reshape(96, 1024), preferred_element_type=f32)
        o_ref[i] = W[0:3, :]


def kernel(image, w1, b1, w2, b2, w3, b3, wd, bd, wl, bl):
    f32 = jnp.float32
    N = image.shape[0]
    Nb1 = _pick_block(N, 64)
    Nb2 = _pick_block(N, 32)

    image = image.astype(f32)

    xp = jnp.pad(image, ((0, 0), (0, 0), (1, 1), (1, 1)))
    taps = [xp[:, :, kh:kh + 32:2, kw:kw + 32:2]
            for kh in range(3) for kw in range(3)]
    col1 = jnp.stack(taps, axis=1)
    col1 = col1.transpose(0, 3, 4, 1, 2).reshape(N, 256, 27)
    col1 = jnp.pad(col1, ((0, 0), (0, 0), (0, 5)))
    col1 = (col1.reshape(N, 4, 4, 4, 4, 32)
            .transpose(0, 2, 4, 1, 3, 5).reshape(N, 256, 32))

    w1f = jnp.pad(w1.transpose(2, 3, 1, 0).reshape(27, 32), ((0, 5), (0, 0)))
    w2f = w2.transpose(2, 3, 1, 0).reshape(288, 64).astype(f32)
    w3f = w3.transpose(2, 3, 1, 0).reshape(576, 128).astype(f32)
    wdr = wd.reshape(128, 128, 16).transpose(2, 1, 0).astype(f32)
    w2h = wl.T.astype(f32)
    b1r = b1.reshape(1, 32).astype(f32)
    b2r = b2.reshape(1, 64).astype(f32)
    b3r = b3.reshape(1, 128).astype(f32)
    bdr = bd.reshape(1, 128).astype(f32)
    blr = bl.reshape(1, 6).astype(f32)

    conv_fn = functools.partial(_conv_head_kernel, Nb=Nb1)
    x3hwc, theta = pl.pallas_call(
        conv_fn,
        out_shape=(jax.ShapeDtypeStruct((N, 16, 128), f32),
                   jax.ShapeDtypeStruct((N, 6), f32)),
        grid_spec=pltpu.PrefetchScalarGridSpec(
            num_scalar_prefetch=0,
            grid=(N // Nb1,),
            in_specs=[
                pl.BlockSpec((Nb1, 256, 32), lambda n: (n, 0, 0)),
                pl.BlockSpec((32, 32), lambda n: (0, 0)),
                pl.BlockSpec((1, 32), lambda n: (0, 0)),
                pl.BlockSpec((288, 64), lambda n: (0, 0)),
                pl.BlockSpec((1, 64), lambda n: (0, 0)),
                pl.BlockSpec((576, 128), lambda n: (0, 0)),
                pl.BlockSpec((1, 128), lambda n: (0, 0)),
                pl.BlockSpec((16, 128, 128), lambda n: (0, 0, 0)),
                pl.BlockSpec((1, 128), lambda n: (0, 0)),
                pl.BlockSpec((128, 6), lambda n: (0, 0)),
                pl.BlockSpec((1, 6), lambda n: (0, 0)),
            ],
            out_specs=(pl.BlockSpec((Nb1, 16, 128), lambda n: (n, 0, 0)),
                       pl.BlockSpec((Nb1, 6), lambda n: (n, 0))),
            scratch_shapes=[
                pltpu.VMEM((Nb1, 20, 5, 128), f32),
                pltpu.VMEM((Nb1, 64, 288), f32),
                pltpu.VMEM((Nb1, 10, 5, 128), f32),
                pltpu.VMEM((Nb1, 16, 576), f32),
            ],
        ),
        compiler_params=pltpu.CompilerParams(
            dimension_semantics=("parallel",)),
    )(col1, w1f, b1r, w2f, b2r, w3f, b3r, wdr, bdr, w2h, blr)

    x_out = x3hwc.transpose(0, 2, 1).reshape(N, 128, 4, 4)

    warp_fn = functools.partial(_warp_kernel, Nb=Nb2)
    img96 = image.reshape(N, 96, 32)
    th3 = theta.reshape(N, 1, 6)
    warped = pl.pallas_call(
        warp_fn,
        out_shape=jax.ShapeDtypeStruct((N, 3, 1024), f32),
        grid_spec=pltpu.PrefetchScalarGridSpec(
            num_scalar_prefetch=0,
            grid=(N // Nb2,),
            in_specs=[
                pl.BlockSpec((Nb2, 96, 32), lambda n: (n, 0, 0)),
                pl.BlockSpec((Nb2, 1, 6), lambda n: (n, 0, 0)),
            ],
            out_specs=pl.BlockSpec((Nb2, 3, 1024), lambda n: (n, 0, 0)),
        ),
        compiler_params=pltpu.CompilerParams(
            dimension_semantics=("parallel",)),
    )(img96, th3)

    return warped.reshape(N, 3, 32, 32), x_out, theta


# speedup vs baseline: 1.1603x; 1.0005x over previous
import functools

import jax
import jax.numpy as jnp
from jax import lax
from jax.experimental import pallas as pl
from jax.experimental.pallas import tpu as pltpu


def _pick_block(n, pref):
    for b in (pref, 32, 16, 8, 4, 2, 1):
        if n % b == 0:
            return b
    return 1


def _conv_head_kernel(col1_ref, w1_ref, b1_ref, w2_ref, b2_ref, w3_ref, b3_ref,
                      wd_ref, bd_ref, wl_ref, bl_ref,
                      x3_ref, th_ref, p2_ref, p3_ref,
                      *, Nb):
    f32 = jnp.float32

    x1 = col1_ref[...].reshape(Nb * 256, 32)
    o1 = jnp.maximum(
        jnp.dot(x1, w1_ref[...], preferred_element_type=f32) + b1_ref[...], 0.0)
    o1r = o1.reshape(Nb, 4, 4, 4, 4, 32)

    zc32 = jnp.zeros((Nb, 4, 1, 32), f32)
    zr128 = jnp.zeros((Nb, 1, 5, 128), f32)
    for a in range(4):
        rh = (a + 3) % 4
        groups = []
        for b in range(4):
            rw = (b + 3) % 4
            blk = o1r[:, rh, rw]
            if b == 0:
                groups.append(jnp.concatenate([zc32, blk], axis=2))
            else:
                groups.append(jnp.concatenate([blk, zc32], axis=2))
        row4 = jnp.concatenate(groups, axis=3)
        if a == 0:
            plane = jnp.concatenate([zr128, row4], axis=1)
        else:
            plane = jnp.concatenate([row4, zr128], axis=1)
        p2_ref[:, a * 5:(a + 1) * 5, :, :] = plane

    dn = (((3,), (0,)), ((), ()))
    o2q = {}
    for ip in range(2):
        for jp in range(2):
            acc = b2_ref[...].reshape(1, 1, 1, 64)
            for kh in range(3):
                ua, um = (2 * ip + kh) % 4, (2 * ip + kh) // 4
                R = p2_ref[:, ua * 5 + um:ua * 5 + um + 4, :, :]
                for gi, vm in (((0, 0),) if jp == 0 else ((1, 0), (2, 1))):
                    acc = acc + lax.dot_general(
                        R[:, :, vm:vm + 4, :], w2_ref[kh * 3 + gi], dn,
                        preferred_element_type=f32)
            o2q[(ip, jp)] = jnp.maximum(acc, 0.0)

    zc64 = jnp.zeros((Nb, 4, 1, 64), f32)
    for a in range(2):
        ip = 1 - a
        b0 = jnp.concatenate([zc64, o2q[(ip, 1)]], axis=2)
        b1 = jnp.concatenate([o2q[(ip, 0)], zc64], axis=2)
        row4 = jnp.concatenate([b0, b1], axis=3)
        if a == 0:
            plane = jnp.concatenate([zr128, row4], axis=1)
        else:
            plane = jnp.concatenate([row4, zr128], axis=1)
        p3_ref[:, a * 5:(a + 1) * 5, :, :] = plane

    acc3 = b3_ref[...].reshape(1, 1, 1, 128)
    for kh in range(3):
        R3 = p3_ref[:, (kh % 2) * 5 + kh // 2:(kh % 2) * 5 + kh // 2 + 4, :, :]
        for vm in range(2):
            acc3 = acc3 + lax.dot_general(
                R3[:, :, vm:vm + 4, :], w3_ref[kh * 2 + vm], dn,
                preferred_element_type=f32)
    o3 = jnp.maximum(acc3, 0.0)
    x3_ref[...] = o3

    acc = bd_ref[...]
    for p in range(16):
        acc = acc + jnp.dot(o3[:, p // 4, p % 4, :], wd_ref[p],
                            preferred_element_type=f32)
    h = jnp.maximum(acc, 0.0)
    th_ref[...] = jnp.dot(h, wl_ref[...], preferred_element_type=f32) + bl_ref[...]


def _warp_kernel(img_ref, th_ref, o_ref, *, Nb):
    f32 = jnp.float32
    pidx = lax.broadcasted_iota(jnp.int32, (1, 1024), 1).astype(f32)
    ohf = jnp.floor(pidx * (1.0 / 32.0))
    owf = pidx - 32.0 * ohf
    xn = (2.0 * owf + 1.0) * (1.0 / 32.0) - 1.0
    yn = (2.0 * ohf + 1.0) * (1.0 / 32.0) - 1.0
    xio = lax.broadcasted_iota(jnp.int32, (32, 1024), 0).astype(f32)
    rr = lax.broadcasted_iota(jnp.int32, (8, 96), 0)
    kk = lax.broadcasted_iota(jnp.int32, (8, 96), 1)
    R = (rr == kk // 32).astype(f32)

    for i in range(Nb):
        trow = th_ref[i]
        gx = trow[:, 0:1] * xn + trow[:, 1:2] * yn + trow[:, 2:3]
        gy = trow[:, 3:4] * xn + trow[:, 4:5] * yn + trow[:, 5:6]
        ix = gx * 16.0 + 15.5
        iy = gy * 16.0 + 15.5
        Bh = jnp.maximum(1.0 - jnp.abs(ix - xio), 0.0)
        Ah = jnp.maximum(1.0 - jnp.abs(iy - xio), 0.0)
        G = jnp.dot(img_ref[i], Bh, preferred_element_type=f32)
        Z = G.reshape(3, 32, 1024) * Ah[None, :, :]
        W = jnp.dot(R, Z.reshape(96, 1024), preferred_element_type=f32)
        o_ref[i] = W[0:3, :]


def kernel(image, w1, b1, w2, b2, w3, b3, wd, bd, wl, bl):
    f32 = jnp.float32
    N = image.shape[0]
    Nb1 = _pick_block(N, 64)
    Nb2 = _pick_block(N, 32)

    image = image.astype(f32)

    xp = jnp.pad(image, ((0, 0), (0, 0), (1, 1), (1, 1)))
    taps = [xp[:, :, kh:kh + 32:2, kw:kw + 32:2]
            for kh in range(3) for kw in range(3)]
    col1 = jnp.stack(taps, axis=1)
    col1 = col1.transpose(0, 3, 4, 1, 2).reshape(N, 256, 27)
    col1 = jnp.pad(col1, ((0, 0), (0, 0), (0, 5)))
    col1 = (col1.reshape(N, 4, 4, 4, 4, 32)
            .transpose(0, 2, 4, 1, 3, 5).reshape(N, 256, 32))

    w1f = jnp.pad(w1.transpose(2, 3, 1, 0).reshape(27, 32), ((0, 5), (0, 0)))
    w2sel = jnp.zeros((9, 128, 64), f32)
    w3sel = jnp.zeros((6, 128, 128), f32)
    for kh in range(3):
        for kw in range(3):
            blkw = w2[:, :, kh, kw].T.astype(f32)
            w2sel = w2sel.at[kh * 3 + 0, kw * 32:(kw + 1) * 32, :].set(blkw)
            if kw < 2:
                w2sel = w2sel.at[kh * 3 + 1, (2 + kw) * 32:(3 + kw) * 32, :].set(blkw)
            else:
                w2sel = w2sel.at[kh * 3 + 2, 0:32, :].set(blkw)
            blkw3 = w3[:, :, kh, kw].T.astype(f32)
            vm, b = kw // 2, kw % 2
            w3sel = w3sel.at[kh * 2 + vm, b * 64:(b + 1) * 64, :].set(blkw3)
    wdr = wd.reshape(128, 128, 16).transpose(2, 1, 0).astype(f32)
    w2h = wl.T.astype(f32)
    b1r = b1.reshape(1, 32).astype(f32)
    b2r = b2.reshape(1, 64).astype(f32)
    b3r = b3.reshape(1, 128).astype(f32)
    bdr = bd.reshape(1, 128).astype(f32)
    blr = bl.reshape(1, 6).astype(f32)

    conv_fn = functools.partial(_conv_head_kernel, Nb=Nb1)
    x3hwc, theta = pl.pallas_call(
        conv_fn,
        out_shape=(jax.ShapeDtypeStruct((N, 4, 4, 128), f32),
                   jax.ShapeDtypeStruct((N, 6), f32)),
        grid_spec=pltpu.PrefetchScalarGridSpec(
            num_scalar_prefetch=0,
            grid=(N // Nb1,),
            in_specs=[
                pl.BlockSpec((Nb1, 256, 32), lambda n: (n, 0, 0)),
                pl.BlockSpec((32, 32), lambda n: (0, 0)),
                pl.BlockSpec((1, 32), lambda n: (0, 0)),
                pl.BlockSpec((9, 128, 64), lambda n: (0, 0, 0)),
                pl.BlockSpec((1, 64), lambda n: (0, 0)),
                pl.BlockSpec((6, 128, 128), lambda n: (0, 0, 0)),
                pl.BlockSpec((1, 128), lambda n: (0, 0)),
                pl.BlockSpec((16, 128, 128), lambda n: (0, 0, 0)),
                pl.BlockSpec((1, 128), lambda n: (0, 0)),
                pl.BlockSpec((128, 6), lambda n: (0, 0)),
                pl.BlockSpec((1, 6), lambda n: (0, 0)),
            ],
            out_specs=(pl.BlockSpec((Nb1, 4, 4, 128), lambda n: (n, 0, 0, 0)),
                       pl.BlockSpec((Nb1, 6), lambda n: (n, 0))),
            scratch_shapes=[
                pltpu.VMEM((Nb1, 20, 5, 128), f32),
                pltpu.VMEM((Nb1, 10, 5, 128), f32),
            ],
        ),
        compiler_params=pltpu.CompilerParams(
            dimension_semantics=("parallel",)),
    )(col1, w1f, b1r, w2sel, b2r, w3sel, b3r, wdr, bdr, w2h, blr)

    x_out = x3hwc.reshape(N, 16, 128).transpose(0, 2, 1).reshape(N, 128, 4, 4)

    warp_fn = functools.partial(_warp_kernel, Nb=Nb2)
    img96 = image.reshape(N, 96, 32)
    th3 = theta.reshape(N, 1, 6)
    warped = pl.pallas_call(
        warp_fn,
        out_shape=jax.ShapeDtypeStruct((N, 3, 1024), f32),
        grid_spec=pltpu.PrefetchScalarGridSpec(
            num_scalar_prefetch=0,
            grid=(N // Nb2,),
            in_specs=[
                pl.BlockSpec((Nb2, 96, 32), lambda n: (n, 0, 0)),
                pl.BlockSpec((Nb2, 1, 6), lambda n: (n, 0, 0)),
            ],
            out_specs=pl.BlockSpec((Nb2, 3, 1024), lambda n: (n, 0, 0)),
        ),
        compiler_params=pltpu.CompilerParams(
            dimension_semantics=("parallel",)),
    )(img96, th3)

    return warped.reshape(N, 3, 32, 32), x_out, theta


# speedup vs baseline: 1.7634x; 1.5199x over previous
import functools

import jax
import jax.numpy as jnp
from jax import lax
from jax.experimental import pallas as pl
from jax.experimental.pallas import tpu as pltpu


def _pick_block(n, pref):
    for b in (pref, 32, 16, 8, 4, 2, 1):
        if n % b == 0:
            return b
    return 1


def _conv_head_kernel(col1_ref, w1_ref, b1_ref, w2_ref, b2_ref, w3_ref, b3_ref,
                      wd_ref, bd_ref, wl_ref, bl_ref,
                      x3_ref, th_ref, p2_ref, p3_ref,
                      *, Nb):
    f32 = jnp.float32

    x1 = col1_ref[...].reshape(Nb * 256, 32)
    o1 = jnp.dot(x1, w1_ref[...], preferred_element_type=f32)
    o1r = o1.reshape(Nb, 4, 4, 4, 4, 32)

    NEG = -1e9
    ng32 = jnp.full((Nb, 4, 1, 32), NEG, f32)
    ngc3 = jnp.full((Nb, 4, 3, 128), NEG, f32)
    ngr = jnp.full((Nb, 1, 8, 128), NEG, f32)
    for a in range(4):
        rh = (a + 3) % 4
        groups = []
        for b in range(4):
            rw = (b + 3) % 4
            blk = o1r[:, rh, rw]
            if b == 0:
                groups.append(jnp.concatenate([ng32, blk], axis=2))
            else:
                groups.append(jnp.concatenate([blk, ng32], axis=2))
        row4 = jnp.concatenate(groups, axis=3)
        row4 = jnp.concatenate([row4, ngc3], axis=2)
        if a == 0:
            plane = jnp.concatenate([ngr, row4], axis=1)
        else:
            plane = jnp.concatenate([row4, ngr], axis=1)
        p2_ref[:, a * 5:(a + 1) * 5, :, :] = jnp.maximum(plane + b1_ref[...], 0.0)

    dn = (((3,), (0,)), ((), ()))
    o2q = {}
    for ip in range(2):
        for jp in range(2):
            acc = b2_ref[...].reshape(1, 1, 1, 64)
            for kh in range(3):
                ua, um = (2 * ip + kh) % 4, (2 * ip + kh) // 4
                R = p2_ref[:, ua * 5 + um:ua * 5 + um + 4, :, :]
                for gi, vm in (((0, 0),) if jp == 0 else ((1, 0), (2, 1))):
                    acc = acc + lax.dot_general(
                        R[:, :, vm:vm + 4, :], w2_ref[kh * 3 + gi], dn,
                        preferred_element_type=f32)
            o2q[(ip, jp)] = acc

    ng64 = jnp.full((Nb, 4, 1, 64), NEG, f32)
    for a in range(2):
        ip = 1 - a
        b0 = jnp.concatenate([ng64, o2q[(ip, 1)]], axis=2)
        b1 = jnp.concatenate([o2q[(ip, 0)], ng64], axis=2)
        row4 = jnp.concatenate([b0, b1], axis=3)
        row4 = jnp.concatenate([row4, ngc3], axis=2)
        if a == 0:
            plane = jnp.concatenate([ngr, row4], axis=1)
        else:
            plane = jnp.concatenate([row4, ngr], axis=1)
        p3_ref[:, a * 5:(a + 1) * 5, :, :] = jnp.maximum(plane, 0.0)

    acc3 = b3_ref[...].reshape(1, 1, 1, 128)
    for kh in range(3):
        R3 = p3_ref[:, (kh % 2) * 5 + kh // 2:(kh % 2) * 5 + kh // 2 + 4, :, :]
        for vm in range(2):
            acc3 = acc3 + lax.dot_general(
                R3[:, :, vm:vm + 4, :], w3_ref[kh * 2 + vm], dn,
                preferred_element_type=f32)
    o3 = jnp.maximum(acc3, 0.0)
    x3_ref[...] = o3

    acc = bd_ref[...]
    for p in range(16):
        acc = acc + jnp.dot(o3[:, p // 4, p % 4, :], wd_ref[p],
                            preferred_element_type=f32)
    h = jnp.maximum(acc, 0.0)
    th_ref[...] = jnp.dot(h, wl_ref[...], preferred_element_type=f32) + bl_ref[...]


def _warp_kernel(img_ref, th_ref, o_ref, *, Nb):
    f32 = jnp.float32
    pidx = lax.broadcasted_iota(jnp.int32, (1, 1024), 1).astype(f32)
    ohf = jnp.floor(pidx * (1.0 / 32.0))
    owf = pidx - 32.0 * ohf
    xn = (2.0 * owf + 1.0) * (1.0 / 32.0) - 1.0
    yn = (2.0 * ohf + 1.0) * (1.0 / 32.0) - 1.0
    xio = lax.broadcasted_iota(jnp.int32, (32, 1024), 0).astype(f32)
    rr = lax.broadcasted_iota(jnp.int32, (8, 96), 0)
    kk = lax.broadcasted_iota(jnp.int32, (8, 96), 1)
    R = (rr == kk // 32).astype(f32)

    th = th_ref[:, 0, :]
    IX = (th[:, 0:1] * xn + th[:, 1:2] * yn + th[:, 2:3]) * 16.0 + 15.5
    IY = (th[:, 3:4] * xn + th[:, 4:5] * yn + th[:, 5:6]) * 16.0 + 15.5

    for i in range(Nb):
        ix = IX[i:i + 1, :]
        iy = IY[i:i + 1, :]
        Bh = jnp.maximum(1.0 - jnp.abs(ix - xio), 0.0)
        Ah = jnp.maximum(1.0 - jnp.abs(iy - xio), 0.0)
        G = jnp.dot(img_ref[i], Bh, preferred_element_type=f32)
        Z = G.reshape(3, 32, 1024) * Ah[None, :, :]
        W = jnp.dot(R, Z.reshape(96, 1024), preferred_element_type=f32)
        o_ref[i] = W[0:3, :]


def kernel(image, w1, b1, w2, b2, w3, b3, wd, bd, wl, bl):
    f32 = jnp.float32
    N = image.shape[0]
    Nb1 = _pick_block(N, 64)
    Nb2 = _pick_block(N, 64)

    image = image.astype(f32)

    eye = jnp.eye(27, dtype=f32).reshape(27, 3, 3, 3).transpose(0, 3, 1, 2)
    eye = jnp.pad(eye, ((0, 5), (0, 0), (0, 0), (0, 0)))
    col1 = lax.conv_general_dilated(
        image, eye, window_strides=(2, 2), padding=((1, 1), (1, 1)),
        dimension_numbers=("NCHW", "OIHW", "NHWC"))
    col1 = (col1.reshape(N, 4, 4, 4, 4, 32)
            .transpose(0, 2, 4, 1, 3, 5).reshape(N, 256, 32))

    w1f = jnp.pad(w1.transpose(2, 3, 1, 0).reshape(27, 32), ((0, 5), (0, 0)))
    w2sel = jnp.zeros((9, 128, 64), f32)
    w3sel = jnp.zeros((6, 128, 128), f32)
    for kh in range(3):
        for kw in range(3):
            blkw = w2[:, :, kh, kw].T.astype(f32)
            w2sel = w2sel.at[kh * 3 + 0, kw * 32:(kw + 1) * 32, :].set(blkw)
            if kw < 2:
                w2sel = w2sel.at[kh * 3 + 1, (2 + kw) * 32:(3 + kw) * 32, :].set(blkw)
            else:
                w2sel = w2sel.at[kh * 3 + 2, 0:32, :].set(blkw)
            blkw3 = w3[:, :, kh, kw].T.astype(f32)
            vm, b = kw // 2, kw % 2
            w3sel = w3sel.at[kh * 2 + vm, b * 64:(b + 1) * 64, :].set(blkw3)
    wdr = wd.reshape(128, 128, 16).transpose(2, 1, 0).astype(f32)
    w2h = wl.T.astype(f32)
    b1r = jnp.tile(b1.astype(f32), 4).reshape(1, 1, 1, 128)
    b2r = b2.reshape(1, 64).astype(f32)
    b3r = b3.reshape(1, 128).astype(f32)
    bdr = bd.reshape(1, 128).astype(f32)
    blr = bl.reshape(1, 6).astype(f32)

    conv_fn = functools.partial(_conv_head_kernel, Nb=Nb1)
    x3hwc, theta = pl.pallas_call(
        conv_fn,
        out_shape=(jax.ShapeDtypeStruct((N, 4, 4, 128), f32),
                   jax.ShapeDtypeStruct((N, 6), f32)),
        grid_spec=pltpu.PrefetchScalarGridSpec(
            num_scalar_prefetch=0,
            grid=(N // Nb1,),
            in_specs=[
                pl.BlockSpec((Nb1, 256, 32), lambda n: (n, 0, 0)),
                pl.BlockSpec((32, 32), lambda n: (0, 0)),
                pl.BlockSpec((1, 1, 1, 128), lambda n: (0, 0, 0, 0)),
                pl.BlockSpec((9, 128, 64), lambda n: (0, 0, 0)),
                pl.BlockSpec((1, 64), lambda n: (0, 0)),
                pl.BlockSpec((6, 128, 128), lambda n: (0, 0, 0)),
                pl.BlockSpec((1, 128), lambda n: (0, 0)),
                pl.BlockSpec((16, 128, 128), lambda n: (0, 0, 0)),
                pl.BlockSpec((1, 128), lambda n: (0, 0)),
                pl.BlockSpec((128, 6), lambda n: (0, 0)),
                pl.BlockSpec((1, 6), lambda n: (0, 0)),
            ],
            out_specs=(pl.BlockSpec((Nb1, 4, 4, 128), lambda n: (n, 0, 0, 0)),
                       pl.BlockSpec((Nb1, 6), lambda n: (n, 0))),
            scratch_shapes=[
                pltpu.VMEM((Nb1, 20, 8, 128), f32),
                pltpu.VMEM((Nb1, 10, 8, 128), f32),
            ],
        ),
        compiler_params=pltpu.CompilerParams(
            dimension_semantics=("parallel",)),
    )(col1, w1f, b1r, w2sel, b2r, w3sel, b3r, wdr, bdr, w2h, blr)

    x_out = x3hwc.reshape(N, 16, 128).transpose(0, 2, 1).reshape(N, 128, 4, 4)

    warp_fn = functools.partial(_warp_kernel, Nb=Nb2)
    img96 = image.reshape(N, 96, 32)
    th3 = theta.reshape(N, 1, 6)
    warped = pl.pallas_call(
        warp_fn,
        out_shape=jax.ShapeDtypeStruct((N, 3, 1024), f32),
        grid_spec=pltpu.PrefetchScalarGridSpec(
            num_scalar_prefetch=0,
            grid=(N // Nb2,),
            in_specs=[
                pl.BlockSpec((Nb2, 96, 32), lambda n: (n, 0, 0)),
                pl.BlockSpec((Nb2, 1, 6), lambda n: (n, 0, 0)),
            ],
            out_specs=pl.BlockSpec((Nb2, 3, 1024), lambda n: (n, 0, 0)),
        ),
        compiler_params=pltpu.CompilerParams(
            dimension_semantics=("parallel",)),
    )(img96, th3)

    return warped.reshape(N, 3, 32, 32), x_out, theta
